# baseline (speedup 1.0000x reference)
import jax
import jax.numpy as jnp
from jax import lax
from jax.experimental import pallas as pl
from jax.experimental.pallas import tpu as pltpu
from jax.experimental.pallas import tpu_sc as plsc

VOCAB = 1000000
DIM = 64
BATCH = 4096
HIST = 50
HIST_PAD = 56
DEPTH = 20

NW = 32
BPW = BATCH // NW
CHUNK_B = 2
CHUNK_ROWS = CHUNK_B * HIST_PAD
NCHUNK = BPW // CHUNK_B
NGROUP = BPW // 16
NBUF = 2

_LOG1P_C = (
    9.9999981056e-01, -4.9997450517e-01, 3.3276187401e-01, -2.4499656640e-01,
    1.7757117522e-01, -1.0785469068e-01, 4.4214724748e-02, -8.5747803338e-03,
)


def _log1p_poly(t):
    acc = jnp.full((16,), _LOG1P_C[-1], jnp.float32)
    for c in reversed(_LOG1P_C[:-1]):
        acc = acc * t + c
    return acc * t


def _fetch_rows(table, idx_vmem, idx_base, n_rows, dst, sem):
    for k in range(n_rows // 16):
        iv = idx_vmem[pl.ds(idx_base + k * 16, 16)]
        for j in range(16):
            r = k * 16 + j
            pltpu.async_copy(table.at[pl.ds(iv[j], 1)], dst.at[pl.ds(r, 1)], sem)


def _sc_kernel(ctx_flat, embeddings, thetas, dummy, dummy2, leaf_out, logp_out,
               idx_all, rv0, rv1, xw_rm, xw_t, idx_v, th_v, node_v, logp_v,
               sem0, sem1, sem_t):
    wid = lax.axis_index("s") * 2 + lax.axis_index("c")
    base = wid * BPW
    iota = lax.iota(jnp.int32, 16)

    pltpu.sync_copy(ctx_flat.at[pl.ds(wid * (BPW * HIST_PAD), BPW * HIST_PAD)],
                    idx_all)

    rvs = (rv0, rv1)
    sems = (sem0, sem1)

    for par in range(NBUF):
        _fetch_rows(embeddings, idx_all, par * CHUNK_ROWS, CHUNK_ROWS,
                    rvs[par], sems[par])

    def p1_body(i, carry):
        for par in range(NBUF):
            g = i * NBUF + par
            rvp = rvs[par]
            semp = sems[par]
            pltpu.make_async_copy(dummy, rvp, semp).wait()
            accs = [
                rvp[b * HIST_PAD, pl.ds(dc * 16, 16)]
                for b in range(CHUNK_B) for dc in range(4)
            ]
            for r in range(1, HIST):
                for b in range(CHUNK_B):
                    for dc in range(4):
                        k = b * 4 + dc
                        accs[k] = accs[k] + rvp[
                            b * HIST_PAD + r, pl.ds(dc * 16, 16)]
            for b in range(CHUNK_B):
                b_local = g * CHUNK_B + b
                for dc in range(4):
                    xw_rm[pl.ds(b_local * DIM + dc * 16, 16)] = accs[b * 4 + dc]
            nxt = g + NBUF

            @pl.when(nxt < NCHUNK)
            def _():
                _fetch_rows(embeddings, idx_all, nxt * CHUNK_ROWS, CHUNK_ROWS,
                            rvp, semp)
        return carry

    lax.fori_loop(0, NCHUNK // NBUF, p1_body, 0)

    for d in range(DIM):
        for bg in range(NGROUP):
            colv = plsc.load_gather(xw_rm, [(iota + bg * 16) * DIM + d])
            xw_t[pl.ds(d * BPW + bg * 16, 16)] = colv


    for bg in range(NGROUP):
        sl = pl.ds(bg * 16, 16)
        node_v[sl] = jnp.zeros((16,), jnp.int32)
        logp_v[sl] = jnp.zeros((16,), jnp.float32)

    def lvl_body(l, carry):
        for bg in range(NGROUP):
            sl = pl.ds(bg * 16, 16)
            idx_v[sl] = jnp.minimum(node_v[sl], VOCAB - 2)
        _fetch_rows(thetas, idx_v, 0, BPW, th_v, sem_t)
        pltpu.make_async_copy(dummy2, th_v, sem_t).wait()
        accs = [jnp.zeros((16,), jnp.float32) for _ in range(NGROUP)]
        for d in range(DIM):
            for bg in range(NGROUP):
                tcol = plsc.load_gather(
                    th_v, [iota + bg * 16, jnp.full((16,), d, jnp.int32)])
                accs[bg] = accs[bg] + tcol * xw_t[pl.ds(d * BPW + bg * 16, 16)]
        for bg in range(NGROUP):
            sl = pl.ds(bg * 16, 16)
            acc = accs[bg]
            right = acc >= 0.0
            t = jnp.exp(-jnp.abs(acc))
            logp_v[sl] = logp_v[sl] - _log1p_poly(t)
            step = jnp.where(right, 1, 0).astype(jnp.int32)
            node_v[sl] = jnp.minimum(node_v[sl] * 2 + 1 + step, 2 * (VOCAB - 1))
        return carry

    lax.fori_loop(0, DEPTH, lvl_body, 0)

    for bg in range(NGROUP):
        sl = pl.ds(bg * 16, 16)
        leaf = node_v[sl] - (VOCAB - 1)
        node_v[sl] = jnp.minimum(jnp.maximum(leaf, 0), VOCAB - 1)
    pltpu.sync_copy(node_v, leaf_out.at[pl.ds(base, BPW)])
    pltpu.sync_copy(logp_v, logp_out.at[pl.ds(base, BPW)])


@jax.jit
def _run(ctx_flat, embeddings, thetas, dummy, dummy2):
    mesh = plsc.VectorSubcoreMesh(core_axis_name="c", subcore_axis_name="s")
    return pl.kernel(
        _sc_kernel,
        mesh=mesh,
        compiler_params=pltpu.CompilerParams(needs_layout_passes=False),
        out_type=[
            jax.ShapeDtypeStruct((BATCH,), jnp.int32),
            jax.ShapeDtypeStruct((BATCH,), jnp.float32),
        ],
        scratch_types=[
            pltpu.VMEM((BPW * HIST_PAD,), jnp.int32),
            pltpu.VMEM((CHUNK_ROWS, DIM), jnp.float32),
            pltpu.VMEM((CHUNK_ROWS, DIM), jnp.float32),
            pltpu.VMEM((BPW * DIM,), jnp.float32),
            pltpu.VMEM((DIM * BPW,), jnp.float32),
            pltpu.VMEM((BPW,), jnp.int32),
            pltpu.VMEM((BPW, DIM), jnp.float32),
            pltpu.VMEM((BPW,), jnp.int32),
            pltpu.VMEM((BPW,), jnp.float32),
            pltpu.SemaphoreType.DMA,
            pltpu.SemaphoreType.DMA,
            pltpu.SemaphoreType.DMA,
        ],
    )(ctx_flat, embeddings, thetas, dummy, dummy2)


def kernel(context, embeddings, thetas):
    ctx = context.astype(jnp.int32)
    ctx_flat = jnp.pad(ctx, ((0, 0), (0, HIST_PAD - HIST))).reshape(-1)
    dummy = jnp.zeros((CHUNK_ROWS, DIM), jnp.float32)
    dummy2 = jnp.zeros((BPW, DIM), jnp.float32)
    leaf, logp = _run(ctx_flat, embeddings, thetas, dummy, dummy2)
    return leaf, logp


# speedup vs baseline: 1.0553x; 1.0553x over previous
import jax
import jax.numpy as jnp
from jax import lax
from jax.experimental import pallas as pl
from jax.experimental.pallas import tpu as pltpu
from jax.experimental.pallas import tpu_sc as plsc

VOCAB = 1000000
DIM = 64
BATCH = 4096
HIST = 50
HIST_PAD = 56
DEPTH = 20

NW = 32
BPW = BATCH // NW
CHUNK_B = 2
CHUNK_ROWS = CHUNK_B * HIST_PAD
NCHUNK = BPW // CHUNK_B
NGROUP = BPW // 16
NBUF = 2

_LOG1P_C = (
    9.9999981056e-01, -4.9997450517e-01, 3.3276187401e-01, -2.4499656640e-01,
    1.7757117522e-01, -1.0785469068e-01, 4.4214724748e-02, -8.5747803338e-03,
)


def _log1p_poly(t):
    acc = jnp.full((16,), _LOG1P_C[-1], jnp.float32)
    for c in reversed(_LOG1P_C[:-1]):
        acc = acc * t + c
    return acc * t


def _fetch_rows(table, idx_vmem, idx_base, n_rows, dst, sem):
    for k in range(n_rows // 16):
        iv = idx_vmem[pl.ds(idx_base + k * 16, 16)]
        pltpu.async_copy(table.at[iv], dst.at[pl.ds(k * 16, 16)], sem)


def _sc_kernel(ctx_flat, embeddings, thetas, dummy, dummy2, leaf_out, logp_out,
               idx_all, rv0, rv1, xw_rm, xw_t, idx_v, th_v, node_v, logp_v,
               sem0, sem1, sem_t):
    wid = lax.axis_index("s") * 2 + lax.axis_index("c")
    base = wid * BPW
    iota = lax.iota(jnp.int32, 16)

    pltpu.sync_copy(ctx_flat.at[pl.ds(wid * (BPW * HIST_PAD), BPW * HIST_PAD)],
                    idx_all)

    rvs = (rv0, rv1)
    sems = (sem0, sem1)

    for par in range(NBUF):
        _fetch_rows(embeddings, idx_all, par * CHUNK_ROWS, CHUNK_ROWS,
                    rvs[par], sems[par])

    def p1_body(i, carry):
        for par in range(NBUF):
            g = i * NBUF + par
            rvp = rvs[par]
            semp = sems[par]
            pltpu.make_async_copy(dummy, rvp, semp).wait()
            accs = [
                rvp[b * HIST_PAD, pl.ds(dc * 16, 16)]
                for b in range(CHUNK_B) for dc in range(4)
            ]
            for r in range(1, HIST):
                for b in range(CHUNK_B):
                    for dc in range(4):
                        k = b * 4 + dc
                        accs[k] = accs[k] + rvp[
                            b * HIST_PAD + r, pl.ds(dc * 16, 16)]
            for b in range(CHUNK_B):
                b_local = g * CHUNK_B + b
                for dc in range(4):
                    xw_rm[pl.ds(b_local * DIM + dc * 16, 16)] = accs[b * 4 + dc]
            nxt = g + NBUF

            @pl.when(nxt < NCHUNK)
            def _():
                _fetch_rows(embeddings, idx_all, nxt * CHUNK_ROWS, CHUNK_ROWS,
                            rvp, semp)
        return carry

    lax.fori_loop(0, NCHUNK // NBUF, p1_body, 0)

    for d in range(DIM):
        for bg in range(NGROUP):
            colv = plsc.load_gather(xw_rm, [(iota + bg * 16) * DIM + d])
            xw_t[pl.ds(d * BPW + bg * 16, 16)] = colv


    for bg in range(NGROUP):
        sl = pl.ds(bg * 16, 16)
        node_v[sl] = jnp.zeros((16,), jnp.int32)
        logp_v[sl] = jnp.zeros((16,), jnp.float32)

    def lvl_body(l, carry):
        for bg in range(NGROUP):
            sl = pl.ds(bg * 16, 16)
            idx_v[sl] = jnp.minimum(node_v[sl], VOCAB - 2)
        _fetch_rows(thetas, idx_v, 0, BPW, th_v, sem_t)
        pltpu.make_async_copy(dummy2, th_v, sem_t).wait()
        accs = [jnp.zeros((16,), jnp.float32) for _ in range(NGROUP)]
        for d in range(DIM):
            for bg in range(NGROUP):
                tcol = plsc.load_gather(
                    th_v, [iota + bg * 16, jnp.full((16,), d, jnp.int32)])
                accs[bg] = accs[bg] + tcol * xw_t[pl.ds(d * BPW + bg * 16, 16)]
        for bg in range(NGROUP):
            sl = pl.ds(bg * 16, 16)
            acc = accs[bg]
            right = acc >= 0.0
            t = jnp.exp(-jnp.abs(acc))
            logp_v[sl] = logp_v[sl] - _log1p_poly(t)
            step = jnp.where(right, 1, 0).astype(jnp.int32)
            node_v[sl] = jnp.minimum(node_v[sl] * 2 + 1 + step, 2 * (VOCAB - 1))
        return carry

    lax.fori_loop(0, DEPTH, lvl_body, 0)

    for bg in range(NGROUP):
        sl = pl.ds(bg * 16, 16)
        leaf = node_v[sl] - (VOCAB - 1)
        node_v[sl] = jnp.minimum(jnp.maximum(leaf, 0), VOCAB - 1)
    pltpu.sync_copy(node_v, leaf_out.at[pl.ds(base, BPW)])
    pltpu.sync_copy(logp_v, logp_out.at[pl.ds(base, BPW)])


@jax.jit
def _run(ctx_flat, embeddings, thetas, dummy, dummy2):
    mesh = plsc.VectorSubcoreMesh(core_axis_name="c", subcore_axis_name="s")
    return pl.kernel(
        _sc_kernel,
        mesh=mesh,
        compiler_params=pltpu.CompilerParams(
            needs_layout_passes=False, use_tc_tiling_on_sc=False),
        out_type=[
            jax.ShapeDtypeStruct((BATCH,), jnp.int32),
            jax.ShapeDtypeStruct((BATCH,), jnp.float32),
        ],
        scratch_types=[
            pltpu.VMEM((BPW * HIST_PAD,), jnp.int32),
            pltpu.VMEM((CHUNK_ROWS, DIM), jnp.float32),
            pltpu.VMEM((CHUNK_ROWS, DIM), jnp.float32),
            pltpu.VMEM((BPW * DIM,), jnp.float32),
            pltpu.VMEM((DIM * BPW,), jnp.float32),
            pltpu.VMEM((BPW,), jnp.int32),
            pltpu.VMEM((BPW, DIM), jnp.float32),
            pltpu.VMEM((BPW,), jnp.int32),
            pltpu.VMEM((BPW,), jnp.float32),
            pltpu.SemaphoreType.DMA,
            pltpu.SemaphoreType.DMA,
            pltpu.SemaphoreType.DMA,
        ],
    )(ctx_flat, embeddings, thetas, dummy, dummy2)


def kernel(context, embeddings, thetas):
    ctx = context.astype(jnp.int32)
    ctx_flat = jnp.pad(ctx, ((0, 0), (0, HIST_PAD - HIST))).reshape(-1)
    dummy = jnp.zeros((CHUNK_ROWS, DIM), jnp.float32)
    dummy2 = jnp.zeros((BPW, DIM), jnp.float32)
    leaf, logp = _run(ctx_flat, embeddings, thetas, dummy, dummy2)
    return leaf, logp
